# baseline (speedup 1.0000x reference)
import functools

import jax
import jax.numpy as jnp
from jax.experimental import pallas as pl
from jax.experimental.pallas import tpu as pltpu

NBRS = 16


def _proj_body(x_ref, w_ref, b_ref, a_ref, c_ref):
    x = x_ref[...]
    w = w_ref[...]
    d = x.shape[1]
    wa = w[:d]
    wb = w[d:]
    a_ref[...] = (
        jnp.dot(x, wa - wb, preferred_element_type=jnp.float32) + b_ref[...]
    )
    c_ref[...] = jnp.dot(x, wb, preferred_element_type=jnp.float32)


def _proj(x, w, b, block_rows=1024):
    npad, d = x.shape
    do = w.shape[1]
    grid = (npad // block_rows,)
    return pl.pallas_call(
        _proj_body,
        grid=grid,
        in_specs=[
            pl.BlockSpec((block_rows, d), lambda i: (i, 0)),
            pl.BlockSpec((2 * d, do), lambda i: (0, 0)),
            pl.BlockSpec((1, do), lambda i: (0, 0)),
        ],
        out_specs=[
            pl.BlockSpec((block_rows, do), lambda i: (i, 0)),
            pl.BlockSpec((block_rows, do), lambda i: (i, 0)),
        ],
        out_shape=[
            jax.ShapeDtypeStruct((npad, do), jnp.float32),
            jax.ShapeDtypeStruct((npad, do), jnp.float32),
        ],
    )(x, w, b[None, :])


def _knn_body(xb_ref, xa_ref, idx_ref, *, nreal, block_rows):
    i = pl.program_id(0)
    xb = xb_ref[...]
    xa = xa_ref[...]
    npad = xa.shape[0]
    sqb = jnp.sum(xb * xb, axis=1, keepdims=True)
    sqa = jnp.sum(xa * xa, axis=1)[None, :]
    prod = jax.lax.dot_general(
        xb, xa, (((1,), (1,)), ((), ())),
        preferred_element_type=jnp.float32,
        precision=jax.lax.Precision.HIGHEST,
    )
    dist = sqb - 2.0 * prod + sqa
    col = jax.lax.broadcasted_iota(jnp.int32, (block_rows, npad), 1)
    row = i * block_rows + jax.lax.broadcasted_iota(
        jnp.int32, (block_rows, npad), 0
    )
    dist = jnp.where((col >= nreal) | (col == row), jnp.inf, dist)
    big = jnp.int32(npad)

    wf = npad // 16
    rounds = 3
    cvals = []
    ccols = []
    v0 = dist
    for r in range(rounds):
        v = v0
        c = col
        w = npad
        while w > wf:
            h = w // 2
            va = v[:, :h]
            vb = v[:, h:w]
            take_a = va <= vb
            v = jnp.where(take_a, va, vb)
            c = jnp.where(take_a, c[:, :h], c[:, h:w])
            w = h
        cvals.append(v)
        ccols.append(c)
        if r < rounds - 1:
            winc = jnp.broadcast_to(
                c[:, None, :], (block_rows, npad // wf, wf)
            ).reshape(block_rows, npad)
            v0 = jnp.where(col == winc, jnp.inf, v0)
    cv = jnp.concatenate(cvals, axis=1)
    cc = jnp.concatenate(ccols, axis=1)
    m = None
    for k in range(NBRS):
        m = jnp.min(cv, axis=1, keepdims=True)
        j = jnp.min(jnp.where(cv <= m, cc, big), axis=1, keepdims=True)
        idx_ref[:, k : k + 1] = j
        cv = jnp.where(cc == j, jnp.inf, cv)
    last = cvals[rounds - 1]
    exhausted = (last <= m) & (last < jnp.inf)
    trig = jnp.max(exhausted.astype(jnp.int32)) > 0

    @pl.when(trig)
    def _fallback():
        prod2 = jax.lax.dot_general(
            xb, xa, (((1,), (1,)), ((), ())),
            preferred_element_type=jnp.float32,
            precision=jax.lax.Precision.HIGHEST,
        )
        d2 = sqb - 2.0 * prod2 + sqa
        d2 = jnp.where((col >= nreal) | (col == row), jnp.inf, d2)
        for k in range(NBRS):
            m = jnp.min(d2, axis=1, keepdims=True)
            j = jnp.min(
                jnp.where(d2 <= m, col, big), axis=1, keepdims=True
            )
            idx_ref[:, k : k + 1] = j
            d2 = jnp.where(col == j, jnp.inf, d2)


def _knn(x, nreal, block_rows=128):
    npad, d = x.shape
    grid = (npad // block_rows,)
    return pl.pallas_call(
        functools.partial(_knn_body, nreal=nreal, block_rows=block_rows),
        grid=grid,
        in_specs=[
            pl.BlockSpec((block_rows, d), lambda i: (i, 0)),
            pl.BlockSpec((npad, d), lambda i: (0, 0)),
        ],
        out_specs=pl.BlockSpec((block_rows, NBRS), lambda i: (i, 0)),
        out_shape=jax.ShapeDtypeStruct((npad, NBRS), jnp.int32),
    )(x, x)


def _agg_body(idx_ref, a_ref, c_ref, o_ref, *, block_rows):
    do = a_ref.shape[1]

    def row(r, _):
        a = a_ref[pl.ds(r, 1), :]
        acc = jnp.zeros((1, do), jnp.float32)
        for k in range(NBRS):
            j = idx_ref[r, k]
            acc = acc + jnp.maximum(a + c_ref[pl.ds(j, 1), :], 0.0)
        o_ref[pl.ds(r, 1), :] = acc * (1.0 / NBRS)
        return 0

    jax.lax.fori_loop(0, block_rows, row, 0)


def _agg(idx, a, c, block_rows=256):
    npad, do = a.shape
    grid = (npad // block_rows,)
    return pl.pallas_call(
        functools.partial(_agg_body, block_rows=block_rows),
        grid=grid,
        in_specs=[
            pl.BlockSpec(
                (block_rows, NBRS), lambda i: (i, 0), memory_space=pltpu.SMEM
            ),
            pl.BlockSpec((block_rows, do), lambda i: (i, 0)),
            pl.BlockSpec((npad, do), lambda i: (0, 0)),
        ],
        out_specs=pl.BlockSpec((block_rows, do), lambda i: (i, 0)),
        out_shape=jax.ShapeDtypeStruct((npad, do), jnp.float32),
    )(idx, a, c)


def _agg_sc(idx, a, c):
    from jax.experimental.pallas import tpu_sc as plsc

    do_real = a.shape[1]
    if do_real < 128:
        a = jnp.pad(a, ((0, 0), (0, 128 - do_real)))
        c = jnp.pad(c, ((0, 0), (0, 128 - do_real)))
    npad, do = a.shape
    info = plsc.get_sparse_core_info()
    nw = info.num_cores * info.num_subcores
    rows_w = npad // nw
    chunk = 8
    nch = rows_w // chunk
    mesh = plsc.VectorSubcoreMesh(core_axis_name="c", subcore_axis_name="s")

    @functools.partial(
        pl.kernel,
        mesh=mesh,
        out_type=jax.ShapeDtypeStruct((npad, do), jnp.float32),
        scratch_types=[
            pltpu.VMEM((chunk * NBRS,), jnp.int32),
            pltpu.VMEM((chunk * NBRS, do), jnp.float32),
            pltpu.VMEM((chunk, do), jnp.float32),
            pltpu.VMEM((chunk, do), jnp.float32),
            pltpu.SemaphoreType.DMA,
        ],
    )
    def body(idx_hbm, a_hbm, c_hbm, out_hbm, idx_v, rows_v, a_v, o_v, sem):
        wid = jax.lax.axis_index("s") * info.num_cores + jax.lax.axis_index(
            "c"
        )
        base = wid * rows_w

        def do_chunk(ci, _):
            r0 = base + ci * chunk
            pltpu.sync_copy(
                idx_hbm.at[pl.ds(r0 * NBRS, chunk * NBRS)], idx_v
            )
            pltpu.async_copy(c_hbm.at[idx_v], rows_v, sem).wait()
            pltpu.sync_copy(a_hbm.at[pl.ds(r0, chunk)], a_v)

            def do_col(dc, _):
                s = dc * 16
                for r in range(chunk):
                    av = a_v[r, pl.ds(s, 16)]
                    acc = jnp.zeros((16,), jnp.float32)
                    for k in range(NBRS):
                        acc = acc + jnp.maximum(
                            av + rows_v[r * NBRS + k, pl.ds(s, 16)], 0.0
                        )
                    o_v[r, pl.ds(s, 16)] = acc * (1.0 / NBRS)
                return 0

            jax.lax.fori_loop(0, do // 16, do_col, 0)
            pltpu.sync_copy(o_v, out_hbm.at[pl.ds(r0, chunk)])
            return 0

        jax.lax.fori_loop(0, nch, do_chunk, 0)

    out = body(idx.reshape(-1), a, c)
    return out[:, :do_real] if do_real < 128 else out


def _mlp_body(
    f0_ref, f1_ref, f2_ref, f3_ref, f4_ref,
    w10_ref, w11_ref, w12_ref, w13_ref, w14_ref, b1_ref,
    w2_ref, b2_ref, w3_ref, b3_ref, o_ref,
):
    h = b1_ref[...]
    for f_ref, w_ref in (
        (f0_ref, w10_ref),
        (f1_ref, w11_ref),
        (f2_ref, w12_ref),
        (f3_ref, w13_ref),
        (f4_ref, w14_ref),
    ):
        h = h + jnp.dot(
            f_ref[...], w_ref[...], preferred_element_type=jnp.float32
        )
    h = jnp.maximum(h, 0.0)
    h = jnp.maximum(
        jnp.dot(h, w2_ref[...], preferred_element_type=jnp.float32)
        + b2_ref[...],
        0.0,
    )
    o_ref[...] = (
        jnp.dot(h, w3_ref[...], preferred_element_type=jnp.float32)
        + b3_ref[...]
    )


def _mlp(feats, w1_parts, b1, w2, b2, w3p, b3p, block_rows=1024):
    npad = feats[0].shape[0]
    d1 = w2.shape[0]
    d2 = w3p.shape[0]
    d3 = w3p.shape[1]
    grid = (npad // block_rows,)
    del d1, d2
    in_specs = [
        pl.BlockSpec((block_rows, f.shape[1]), lambda i: (i, 0)) for f in feats
    ] + [pl.BlockSpec(w.shape, lambda i: (0, 0)) for w in w1_parts] + [
        pl.BlockSpec((1, b1.shape[0]), lambda i: (0, 0)),
        pl.BlockSpec(w2.shape, lambda i: (0, 0)),
        pl.BlockSpec((1, b2.shape[0]), lambda i: (0, 0)),
        pl.BlockSpec(w3p.shape, lambda i: (0, 0)),
        pl.BlockSpec((1, d3), lambda i: (0, 0)),
    ]
    return pl.pallas_call(
        _mlp_body,
        grid=grid,
        in_specs=in_specs,
        out_specs=pl.BlockSpec((block_rows, d3), lambda i: (i, 0)),
        out_shape=jax.ShapeDtypeStruct((npad, d3), jnp.float32),
    )(*feats, *w1_parts, b1[None, :], w2, b2[None, :], w3p, b3p[None, :])


def _dgcnn(x, W1, b1, W2, b2, W3, b3, W4, b4, Wm1, bm1, Wm2, bm2, Wm3, bm3):
    nreal = x.shape[0]
    npad = ((nreal + 1023) // 1024) * 1024
    xp = jnp.pad(x, ((0, npad - nreal), (0, 0)))
    feats = [xp]
    cur = xp
    for w, b in ((W1, b1), (W2, b2), (W3, b3), (W4, b4)):
        a, c = _proj(cur, w, b)
        idx = _knn(cur, nreal)
        cur = _agg_sc(idx, a, c)
        feats.append(cur)
    parts = []
    off = 0
    for f in feats:
        dk = 3 if f is feats[0] else f.shape[1]
        parts.append(Wm1[off : off + dk])
        off += dk
    parts[0] = jnp.pad(parts[0], ((0, feats[0].shape[1] - 3), (0, 0)))
    d3 = 128
    w3p = jnp.pad(Wm3, ((0, 0), (0, d3 - Wm3.shape[1])))
    b3p = jnp.pad(bm3, ((0, d3 - bm3.shape[0]),))
    out = _mlp(feats, parts, bm1, Wm2, bm2, w3p, b3p)
    return out[:nreal, : Wm3.shape[1]]


def kernel(x, W1, b1, W2, b2, W3, b3, W4, b4, Wm1, bm1, Wm2, bm2, Wm3, bm3):
    return _dgcnn(
        x, W1, b1, W2, b2, W3, b3, W4, b4, Wm1, bm1, Wm2, bm2, Wm3, bm3
    )


# speedup vs baseline: 1.2943x; 1.2943x over previous
import functools

import jax
import jax.numpy as jnp
from jax.experimental import pallas as pl
from jax.experimental.pallas import tpu as pltpu

NBRS = 16


def _proj_body(x_ref, w_ref, b_ref, a_ref, c_ref):
    x = x_ref[...]
    w = w_ref[...]
    d = x.shape[1]
    wa = w[:d]
    wb = w[d:]
    a_ref[...] = (
        jnp.dot(x, wa - wb, preferred_element_type=jnp.float32) + b_ref[...]
    )
    c_ref[...] = jnp.dot(x, wb, preferred_element_type=jnp.float32)


def _proj(x, w, b, block_rows=1024):
    npad, d = x.shape
    do = w.shape[1]
    grid = (npad // block_rows,)
    return pl.pallas_call(
        _proj_body,
        grid=grid,
        in_specs=[
            pl.BlockSpec((block_rows, d), lambda i: (i, 0)),
            pl.BlockSpec((2 * d, do), lambda i: (0, 0)),
            pl.BlockSpec((1, do), lambda i: (0, 0)),
        ],
        out_specs=[
            pl.BlockSpec((block_rows, do), lambda i: (i, 0)),
            pl.BlockSpec((block_rows, do), lambda i: (i, 0)),
        ],
        out_shape=[
            jax.ShapeDtypeStruct((npad, do), jnp.float32),
            jax.ShapeDtypeStruct((npad, do), jnp.float32),
        ],
    )(x, w, b[None, :])


def _knn_body(xb_ref, xa_ref, idx_ref, *, nreal, block_rows):
    i = pl.program_id(0)
    xb = xb_ref[...]
    xa = xa_ref[...]
    npad = xa.shape[0]
    sqb = jnp.sum(xb * xb, axis=1, keepdims=True)
    xa2 = xa * xa
    ones = jnp.ones((1, xa.shape[1]), jnp.float32)
    sqa = jax.lax.dot_general(
        ones, xa2, (((1,), (1,)), ((), ())), preferred_element_type=jnp.float32
    )
    prod = jax.lax.dot_general(
        xb, xa, (((1,), (1,)), ((), ())), preferred_element_type=jnp.float32
    )
    dist = sqb - 2.0 * prod + sqa
    col = jax.lax.broadcasted_iota(jnp.int32, (block_rows, npad), 1)
    row = i * block_rows + jax.lax.broadcasted_iota(
        jnp.int32, (block_rows, npad), 0
    )
    dist = jnp.where((col >= nreal) | (col == row), jnp.inf, dist)
    big = jnp.int32(npad)

    wf = npad // 16
    rounds = 3
    cvals = []
    ccols = []
    v0 = dist
    for r in range(rounds):
        v = v0
        c = col
        w = npad
        while w > wf:
            h = w // 2
            va = v[:, :h]
            vb = v[:, h:w]
            take_a = va <= vb
            v = jnp.where(take_a, va, vb)
            c = jnp.where(take_a, c[:, :h], c[:, h:w])
            w = h
        cvals.append(v)
        ccols.append(c)
        if r < rounds - 1:
            winc = jnp.broadcast_to(
                c[:, None, :], (block_rows, npad // wf, wf)
            ).reshape(block_rows, npad)
            v0 = jnp.where(col == winc, jnp.inf, v0)
    cv = jnp.concatenate(cvals, axis=1)
    cc = jnp.concatenate(ccols, axis=1)
    m = None
    for k in range(NBRS):
        m = jnp.min(cv, axis=1, keepdims=True)
        j = jnp.min(jnp.where(cv <= m, cc, big), axis=1, keepdims=True)
        idx_ref[:, k : k + 1] = j
        cv = jnp.where(cc == j, jnp.inf, cv)
    last = cvals[rounds - 1]
    exhausted = (last <= m) & (last < jnp.inf)
    trig = jnp.max(exhausted.astype(jnp.int32)) > 0

    @pl.when(trig)
    def _fallback():
        prod2 = jax.lax.dot_general(
            xb, xa, (((1,), (1,)), ((), ())),
            preferred_element_type=jnp.float32,
        )
        d2 = sqb - 2.0 * prod2 + sqa
        d2 = jnp.where((col >= nreal) | (col == row), jnp.inf, d2)
        for k in range(NBRS):
            m = jnp.min(d2, axis=1, keepdims=True)
            j = jnp.min(
                jnp.where(d2 <= m, col, big), axis=1, keepdims=True
            )
            idx_ref[:, k : k + 1] = j
            d2 = jnp.where(col == j, jnp.inf, d2)


def _knn(x, nreal, block_rows=128):
    npad, d = x.shape
    grid = (npad // block_rows,)
    return pl.pallas_call(
        functools.partial(_knn_body, nreal=nreal, block_rows=block_rows),
        grid=grid,
        in_specs=[
            pl.BlockSpec((block_rows, d), lambda i: (i, 0)),
            pl.BlockSpec((npad, d), lambda i: (0, 0)),
        ],
        out_specs=pl.BlockSpec((block_rows, NBRS), lambda i: (i, 0)),
        out_shape=jax.ShapeDtypeStruct((npad, NBRS), jnp.int32),
    )(x, x)


def _agg_body(idx_ref, a_ref, c_ref, o_ref, *, block_rows):
    do = a_ref.shape[1]

    def row(r, _):
        a = a_ref[pl.ds(r, 1), :]
        acc = jnp.zeros((1, do), jnp.float32)
        for k in range(NBRS):
            j = idx_ref[r, k]
            acc = acc + jnp.maximum(a + c_ref[pl.ds(j, 1), :], 0.0)
        o_ref[pl.ds(r, 1), :] = acc * (1.0 / NBRS)
        return 0

    jax.lax.fori_loop(0, block_rows, row, 0)


def _agg(idx, a, c, block_rows=256):
    npad, do = a.shape
    grid = (npad // block_rows,)
    return pl.pallas_call(
        functools.partial(_agg_body, block_rows=block_rows),
        grid=grid,
        in_specs=[
            pl.BlockSpec(
                (block_rows, NBRS), lambda i: (i, 0), memory_space=pltpu.SMEM
            ),
            pl.BlockSpec((block_rows, do), lambda i: (i, 0)),
            pl.BlockSpec((npad, do), lambda i: (0, 0)),
        ],
        out_specs=pl.BlockSpec((block_rows, do), lambda i: (i, 0)),
        out_shape=jax.ShapeDtypeStruct((npad, do), jnp.float32),
    )(idx, a, c)


def _agg_sc(idx, a, c):
    from jax.experimental.pallas import tpu_sc as plsc

    do_real = a.shape[1]
    if do_real < 128:
        a = jnp.pad(a, ((0, 0), (0, 128 - do_real)))
        c = jnp.pad(c, ((0, 0), (0, 128 - do_real)))
    npad, do = a.shape
    info = plsc.get_sparse_core_info()
    nw = info.num_cores * info.num_subcores
    rows_w = npad // nw
    chunk = 8
    nch = rows_w // chunk
    mesh = plsc.VectorSubcoreMesh(core_axis_name="c", subcore_axis_name="s")

    @functools.partial(
        pl.kernel,
        mesh=mesh,
        out_type=jax.ShapeDtypeStruct((npad, do), jnp.float32),
        scratch_types=[
            pltpu.VMEM((chunk * NBRS,), jnp.int32),
            pltpu.VMEM((chunk * NBRS, do), jnp.float32),
            pltpu.VMEM((chunk, do), jnp.float32),
            pltpu.VMEM((chunk, do), jnp.float32),
            pltpu.SemaphoreType.DMA,
        ],
    )
    def body(idx_hbm, a_hbm, c_hbm, out_hbm, idx_v, rows_v, a_v, o_v, sem):
        wid = jax.lax.axis_index("s") * info.num_cores + jax.lax.axis_index(
            "c"
        )
        base = wid * rows_w

        def do_chunk(ci, _):
            r0 = base + ci * chunk
            pltpu.sync_copy(
                idx_hbm.at[pl.ds(r0 * NBRS, chunk * NBRS)], idx_v
            )
            pltpu.async_copy(c_hbm.at[idx_v], rows_v, sem).wait()
            pltpu.sync_copy(a_hbm.at[pl.ds(r0, chunk)], a_v)

            def do_col(dc, _):
                s = dc * 16
                for r in range(chunk):
                    av = a_v[r, pl.ds(s, 16)]
                    acc = jnp.zeros((16,), jnp.float32)
                    for k in range(NBRS):
                        acc = acc + jnp.maximum(
                            av + rows_v[r * NBRS + k, pl.ds(s, 16)], 0.0
                        )
                    o_v[r, pl.ds(s, 16)] = acc * (1.0 / NBRS)
                return 0

            jax.lax.fori_loop(0, do // 16, do_col, 0)
            pltpu.sync_copy(o_v, out_hbm.at[pl.ds(r0, chunk)])
            return 0

        jax.lax.fori_loop(0, nch, do_chunk, 0)

    out = body(idx.reshape(-1), a, c)
    return out[:, :do_real] if do_real < 128 else out


def _mlp_body(
    f0_ref, f1_ref, f2_ref, f3_ref, f4_ref,
    w10_ref, w11_ref, w12_ref, w13_ref, w14_ref, b1_ref,
    w2_ref, b2_ref, w3_ref, b3_ref, o_ref,
):
    h = b1_ref[...]
    for f_ref, w_ref in (
        (f0_ref, w10_ref),
        (f1_ref, w11_ref),
        (f2_ref, w12_ref),
        (f3_ref, w13_ref),
        (f4_ref, w14_ref),
    ):
        h = h + jnp.dot(
            f_ref[...], w_ref[...], preferred_element_type=jnp.float32
        )
    h = jnp.maximum(h, 0.0)
    h = jnp.maximum(
        jnp.dot(h, w2_ref[...], preferred_element_type=jnp.float32)
        + b2_ref[...],
        0.0,
    )
    o_ref[...] = (
        jnp.dot(h, w3_ref[...], preferred_element_type=jnp.float32)
        + b3_ref[...]
    )


def _mlp(feats, w1_parts, b1, w2, b2, w3p, b3p, block_rows=1024):
    npad = feats[0].shape[0]
    d1 = w2.shape[0]
    d2 = w3p.shape[0]
    d3 = w3p.shape[1]
    grid = (npad // block_rows,)
    del d1, d2
    in_specs = [
        pl.BlockSpec((block_rows, f.shape[1]), lambda i: (i, 0)) for f in feats
    ] + [pl.BlockSpec(w.shape, lambda i: (0, 0)) for w in w1_parts] + [
        pl.BlockSpec((1, b1.shape[0]), lambda i: (0, 0)),
        pl.BlockSpec(w2.shape, lambda i: (0, 0)),
        pl.BlockSpec((1, b2.shape[0]), lambda i: (0, 0)),
        pl.BlockSpec(w3p.shape, lambda i: (0, 0)),
        pl.BlockSpec((1, d3), lambda i: (0, 0)),
    ]
    return pl.pallas_call(
        _mlp_body,
        grid=grid,
        in_specs=in_specs,
        out_specs=pl.BlockSpec((block_rows, d3), lambda i: (i, 0)),
        out_shape=jax.ShapeDtypeStruct((npad, d3), jnp.float32),
    )(*feats, *w1_parts, b1[None, :], w2, b2[None, :], w3p, b3p[None, :])


def _dgcnn(x, W1, b1, W2, b2, W3, b3, W4, b4, Wm1, bm1, Wm2, bm2, Wm3, bm3):
    nreal = x.shape[0]
    npad = ((nreal + 1023) // 1024) * 1024
    xp = jnp.pad(x, ((0, npad - nreal), (0, 0)))
    feats = [xp]
    cur = xp
    for w, b in ((W1, b1), (W2, b2), (W3, b3), (W4, b4)):
        a, c = _proj(cur, w, b)
        idx = _knn(cur, nreal)
        cur = _agg_sc(idx, a, c)
        feats.append(cur)
    parts = []
    off = 0
    for f in feats:
        dk = 3 if f is feats[0] else f.shape[1]
        parts.append(Wm1[off : off + dk])
        off += dk
    parts[0] = jnp.pad(parts[0], ((0, feats[0].shape[1] - 3), (0, 0)))
    d3 = 128
    w3p = jnp.pad(Wm3, ((0, 0), (0, d3 - Wm3.shape[1])))
    b3p = jnp.pad(bm3, ((0, d3 - bm3.shape[0]),))
    out = _mlp(feats, parts, bm1, Wm2, bm2, w3p, b3p)
    return out[:nreal, : Wm3.shape[1]]


def kernel(x, W1, b1, W2, b2, W3, b3, W4, b4, Wm1, bm1, Wm2, bm2, Wm3, bm3):
    return _dgcnn(
        x, W1, b1, W2, b2, W3, b3, W4, b4, Wm1, bm1, Wm2, bm2, Wm3, bm3
    )
